# baseline (speedup 1.0000x reference)
import functools

import jax
import jax.numpy as jnp
from jax import lax
from jax.experimental import pallas as pl
from jax.experimental.pallas import tpu as pltpu
from jax.experimental.pallas import tpu_sc as plsc

INPUT_DIM = 4096
NUM_EXPERTS = 64
K = 8
BLK = 1024
NUM_CORES = 2
NUM_SUBCORES = 16
LANES = 16
NUM_WORKERS = NUM_CORES * NUM_SUBCORES
SC_CHUNK = 512

_NET8 = [
    (0, 1), (2, 3), (4, 5), (6, 7),
    (0, 2), (1, 3), (4, 6), (5, 7),
    (1, 2), (5, 6),
    (0, 4), (1, 5), (2, 6), (3, 7),
    (2, 4), (3, 5),
    (1, 2), (3, 4), (5, 6),
]
_BITONIC8 = [
    (0, 4), (1, 5), (2, 6), (3, 7),
    (0, 2), (1, 3), (4, 6), (5, 7),
    (0, 1), (2, 3), (4, 5), (6, 7),
]


def _sort8_desc(r):
    r = list(r)
    for i, j in _NET8:
        hi = jnp.maximum(r[i], r[j])
        r[j] = jnp.minimum(r[i], r[j])
        r[i] = hi
    return r


def _merge_top8(a, b):
    c = [jnp.maximum(a[i], b[7 - i]) for i in range(8)]
    for i, j in _BITONIC8:
        hi = jnp.maximum(c[i], c[j])
        c[j] = jnp.minimum(c[i], c[j])
        c[i] = hi
    return c


def _kth_largest(vals):
    gs = [_sort8_desc(vals[8 * k:8 * k + 8]) for k in range(8)]
    m1 = [_merge_top8(gs[2 * k], gs[2 * k + 1]) for k in range(4)]
    m2 = [_merge_top8(m1[2 * k], m1[2 * k + 1]) for k in range(2)]
    c = [jnp.maximum(m2[0][i], m2[1][7 - i]) for i in range(8)]
    t01 = jnp.minimum(c[0], c[1])
    t23 = jnp.minimum(c[2], c[3])
    t45 = jnp.minimum(c[4], c[5])
    t67 = jnp.minimum(c[6], c[7])
    return jnp.minimum(jnp.minimum(t01, t23), jnp.minimum(t45, t67))


def _tc_body(x_ref, w_ref, b_ref, o_ref):
    logits = lax.dot_general(
        w_ref[...], x_ref[...],
        (((1,), (1,)), ((), ())),
        preferred_element_type=jnp.float32,
    ) + b_ref[...]
    m = jnp.max(logits, axis=0, keepdims=True)
    e = jnp.exp(logits - m)
    o_ref[...] = e / jnp.sum(e, axis=0, keepdims=True)


def _sc_body(rwT_hbm, out_hbm, rw_v, out_v, sem):
    wid = lax.axis_index("s") * NUM_CORES + lax.axis_index("c")
    tokens_per_worker = rwT_hbm.shape[1] // NUM_WORKERS
    base = wid * tokens_per_worker
    pltpu.async_copy(
        rwT_hbm.at[:, pl.ds(base, tokens_per_worker)], rw_v, sem).wait()

    for chunk in range(tokens_per_worker // SC_CHUNK):
        coff = chunk * SC_CHUNK

        @plsc.parallel_loop(0, SC_CHUNK // LANES)
        def group_body(g):
            off = coff + g * LANES
            vals = [rw_v[e, pl.ds(off, LANES)] for e in range(NUM_EXPERTS)]
            thresh = _kth_largest(vals)
            for e in range(NUM_EXPERTS):
                out_v[e, pl.ds(off - coff, LANES)] = jnp.where(
                    vals[e] >= thresh, vals[e], 0.0)
        pltpu.sync_copy(
            out_v, out_hbm.at[:, pl.ds(base + coff, SC_CHUNK)])


@jax.jit
def kernel(x, W, b):
    B, S, D = x.shape
    E = W.shape[0]
    N = B * S
    x2 = x.reshape(N, D)

    rwT = pl.pallas_call(
        _tc_body,
        grid=(N // BLK,),
        in_specs=[
            pl.BlockSpec((BLK, D), lambda i: (i, 0)),
            pl.BlockSpec((E, D), lambda i: (0, 0)),
            pl.BlockSpec((E, 1), lambda i: (0, 0)),
        ],
        out_specs=pl.BlockSpec((E, BLK), lambda i: (0, i)),
        out_shape=jax.ShapeDtypeStruct((E, N), jnp.float32),
    )(x2, W, b.reshape(E, 1))

    mesh = plsc.VectorSubcoreMesh(
        core_axis_name="c", subcore_axis_name="s",
        num_cores=NUM_CORES, num_subcores=NUM_SUBCORES)
    sc_topk = functools.partial(
        pl.kernel,
        out_type=jax.ShapeDtypeStruct((E, N), jnp.float32),
        mesh=mesh,
        scratch_types=[
            pltpu.VMEM((E, N // NUM_WORKERS), jnp.float32),
            pltpu.VMEM((E, SC_CHUNK), jnp.float32),
            pltpu.SemaphoreType.DMA,
        ],
        compiler_params=pltpu.CompilerParams(needs_layout_passes=False),
    )(_sc_body)
    outT = sc_topk(rwT)
    return outT.T.reshape(B, S, E)


# speedup vs baseline: 1.0681x; 1.0681x over previous
import functools

import jax
import jax.numpy as jnp
from jax import lax
from jax.experimental import pallas as pl
from jax.experimental.pallas import tpu as pltpu
from jax.experimental.pallas import tpu_sc as plsc

INPUT_DIM = 4096
NUM_EXPERTS = 64
K = 8
BLK = 1024
NUM_CORES = 2
NUM_SUBCORES = 16
LANES = 16
NUM_WORKERS = NUM_CORES * NUM_SUBCORES
SC_CHUNK = 512

_NET8 = [
    (0, 1), (2, 3), (4, 5), (6, 7),
    (0, 2), (1, 3), (4, 6), (5, 7),
    (1, 2), (5, 6),
    (0, 4), (1, 5), (2, 6), (3, 7),
    (2, 4), (3, 5),
    (1, 2), (3, 4), (5, 6),
]
_BITONIC8 = [
    (0, 4), (1, 5), (2, 6), (3, 7),
    (0, 2), (1, 3), (4, 6), (5, 7),
    (0, 1), (2, 3), (4, 5), (6, 7),
]


def _sort8_desc(r):
    r = list(r)
    for i, j in _NET8:
        hi = jnp.maximum(r[i], r[j])
        r[j] = jnp.minimum(r[i], r[j])
        r[i] = hi
    return r


def _merge_top8(a, b):
    c = [jnp.maximum(a[i], b[7 - i]) for i in range(8)]
    for i, j in _BITONIC8:
        hi = jnp.maximum(c[i], c[j])
        c[j] = jnp.minimum(c[i], c[j])
        c[i] = hi
    return c


def _kth_largest(vals):
    gs = [_sort8_desc(vals[8 * k:8 * k + 8]) for k in range(8)]
    m1 = [_merge_top8(gs[2 * k], gs[2 * k + 1]) for k in range(4)]
    m2 = [_merge_top8(m1[2 * k], m1[2 * k + 1]) for k in range(2)]
    c = [jnp.maximum(m2[0][i], m2[1][7 - i]) for i in range(8)]
    t01 = jnp.minimum(c[0], c[1])
    t23 = jnp.minimum(c[2], c[3])
    t45 = jnp.minimum(c[4], c[5])
    t67 = jnp.minimum(c[6], c[7])
    return jnp.minimum(jnp.minimum(t01, t23), jnp.minimum(t45, t67))


def _tc_body(x_ref, w_ref, b_ref, o_ref):
    logits = lax.dot_general(
        w_ref[...], x_ref[...],
        (((1,), (1,)), ((), ())),
        preferred_element_type=jnp.float32,
    ) + b_ref[...]
    m = jnp.max(logits, axis=0, keepdims=True)
    e = jnp.exp(logits - m)
    o_ref[...] = e / jnp.sum(e, axis=0, keepdims=True)


def _sc_body(rwT_hbm, out_hbm, rw_v, out_v, sem):
    wid = lax.axis_index("s") * NUM_CORES + lax.axis_index("c")
    tokens_per_worker = rwT_hbm.shape[1] // NUM_WORKERS
    base = wid * tokens_per_worker
    pltpu.async_copy(
        rwT_hbm.at[:, pl.ds(base, tokens_per_worker)], rw_v, sem).wait()

    for chunk in range(tokens_per_worker // SC_CHUNK):
        coff = chunk * SC_CHUNK

        def group_body(g, _):
            off = coff + g * LANES
            vals = [rw_v[e, pl.ds(off, LANES)] for e in range(NUM_EXPERTS)]
            thresh = _kth_largest(vals)
            for e in range(NUM_EXPERTS):
                out_v[e, pl.ds(off - coff, LANES)] = jnp.where(
                    vals[e] >= thresh, vals[e], 0.0)
            return 0

        lax.fori_loop(0, SC_CHUNK // LANES, group_body, 0)
        pltpu.sync_copy(
            out_v, out_hbm.at[:, pl.ds(base + coff, SC_CHUNK)])


@jax.jit
def kernel(x, W, b):
    B, S, D = x.shape
    E = W.shape[0]
    N = B * S
    x2 = x.reshape(N, D)

    rwT = pl.pallas_call(
        _tc_body,
        grid=(N // BLK,),
        in_specs=[
            pl.BlockSpec((BLK, D), lambda i: (i, 0)),
            pl.BlockSpec((E, D), lambda i: (0, 0)),
            pl.BlockSpec((E, 1), lambda i: (0, 0)),
        ],
        out_specs=pl.BlockSpec((E, BLK), lambda i: (0, i)),
        out_shape=jax.ShapeDtypeStruct((E, N), jnp.float32),
    )(x2, W, b.reshape(E, 1))

    mesh = plsc.VectorSubcoreMesh(
        core_axis_name="c", subcore_axis_name="s",
        num_cores=NUM_CORES, num_subcores=NUM_SUBCORES)
    sc_topk = functools.partial(
        pl.kernel,
        out_type=jax.ShapeDtypeStruct((E, N), jnp.float32),
        mesh=mesh,
        scratch_types=[
            pltpu.VMEM((E, N // NUM_WORKERS), jnp.float32),
            pltpu.VMEM((E, SC_CHUNK), jnp.float32),
            pltpu.SemaphoreType.DMA,
        ],
        compiler_params=pltpu.CompilerParams(needs_layout_passes=False),
    )(_sc_body)
    outT = sc_topk(rwT)
    return outT.T.reshape(B, S, E)
